# baseline (speedup 1.0000x reference)
import functools

import jax
import jax.numpy as jnp
from jax import lax
from jax.experimental import pallas as pl
from jax.experimental.pallas import tpu as pltpu
from jax.experimental.pallas import tpu_sc as plsc

N_NODES = 10000
N_EDGES = 320000
NC, NS = 2, 16
NT = NC * NS
CHUNK = 128
NCHUNK_D = 80
EPAD_D = NT * NCHUNK_D * CHUNK
NCHUNK_A = 158
EPAD_A = NS * NCHUNK_A * CHUNK
ROWS = 10112
RPT = ROWS // NS
NPT = N_NODES // NS
PAD_DST = N_NODES + 8
_DLANES = 8
_MESH = plsc.VectorSubcoreMesh(core_axis_name="c", subcore_axis_name="s")


def _make_sc_agg(F):
    F2 = F // 2

    @functools.partial(
        pl.kernel,
        out_type=jax.ShapeDtypeStruct((ROWS, F), jnp.float32),
        mesh=_MESH,
        compiler_params=pltpu.CompilerParams(use_tc_tiling_on_sc=False),
        scratch_types=[
            pltpu.VMEM((NCHUNK_A, CHUNK), jnp.int32),
            pltpu.VMEM((NCHUNK_A, CHUNK), jnp.int32),
            pltpu.VMEM((CHUNK, F2), jnp.float32),
            pltpu.VMEM((CHUNK, F2), jnp.float32),
            pltpu.VMEM_SHARED((N_NODES, F2), jnp.float32),
            pltpu.VMEM_SHARED((ROWS, F2), jnp.float32),
            pltpu.SemaphoreType.DMA,
            pltpu.SemaphoreType.DMA,
        ],
    )
    def agg(g_hbm, src_hbm, dst_hbm, zero_hbm, out_hbm,
            src_v, dst_v, gb0, gb1, gsh, acc, sem0, sem1):
        c = lax.axis_index("c")
        s = lax.axis_index("s")
        col0 = c * F2
        pltpu.sync_copy(src_hbm.at[s], src_v)
        pltpu.sync_copy(dst_hbm.at[s], dst_v)
        g0 = s * NPT
        pltpu.sync_copy(g_hbm.at[pl.ds(g0, NPT), pl.ds(col0, F2)],
                        gsh.at[pl.ds(g0, NPT)])
        r0 = s * RPT
        pltpu.sync_copy(zero_hbm.at[pl.ds(r0, RPT)], acc.at[pl.ds(r0, RPT)])
        plsc.subcore_barrier()

        gbufs = (gb0, gb1)
        sems = (sem0, sem1)
        for b in range(2):
            pltpu.async_copy(gsh.at[src_v.at[b]], gbufs[b], sems[b])

        @pl.loop(0, NCHUNK_A - 2, step=2)
        def _(j0):
            for b in range(2):
                j = j0 + b
                pltpu.make_async_copy(gsh.at[src_v.at[j]], gbufs[b], sems[b]).wait()
                pltpu.sync_copy(gbufs[b], acc.at[dst_v.at[j]], add=True)
                pltpu.async_copy(gsh.at[src_v.at[j + 2]], gbufs[b], sems[b])

        for b in range(2):
            j = NCHUNK_A - 2 + b
            pltpu.make_async_copy(gsh.at[src_v.at[j]], gbufs[b], sems[b]).wait()
            pltpu.sync_copy(gbufs[b], acc.at[dst_v.at[j]], add=True)

        plsc.subcore_barrier()
        pltpu.sync_copy(acc.at[pl.ds(r0, RPT)],
                        out_hbm.at[pl.ds(r0, RPT), pl.ds(col0, F2)])

    return agg


_AGG = {f: _make_sc_agg(f) for f in (64, 32, 16)}


@functools.partial(
    pl.kernel,
    out_type=jax.ShapeDtypeStruct((ROWS, 2 * _DLANES), jnp.float32),
    mesh=_MESH,
    compiler_params=pltpu.CompilerParams(use_tc_tiling_on_sc=False),
    scratch_types=[
        pltpu.VMEM((NCHUNK_D, CHUNK), jnp.int32),
        pltpu.VMEM((CHUNK, _DLANES), jnp.float32),
        pltpu.VMEM_SHARED((ROWS, _DLANES), jnp.float32),
        pltpu.SemaphoreType.DMA,
    ],
)
def _sc_deg(dst_hbm, ones_hbm, zero_hbm, out_hbm, dst_v, ones_v, acc, sem):
    c = lax.axis_index("c")
    s = lax.axis_index("s")
    wid = c * NS + s
    pltpu.sync_copy(dst_hbm.at[wid], dst_v)
    pltpu.sync_copy(ones_hbm, ones_v)
    r0 = s * RPT
    pltpu.sync_copy(zero_hbm.at[pl.ds(r0, RPT)], acc.at[pl.ds(r0, RPT)])
    plsc.subcore_barrier()

    @pl.loop(0, NCHUNK_D)
    def _(j):
        pltpu.async_copy(ones_v, acc.at[dst_v.at[j]], sem, add=True)

    @pl.loop(0, NCHUNK_D)
    def _(j):
        pltpu.make_async_copy(ones_v, acc.at[dst_v.at[j]], sem).wait()

    plsc.subcore_barrier()
    pltpu.sync_copy(acc.at[pl.ds(r0, RPT)],
                    out_hbm.at[pl.ds(r0, RPT), pl.ds(c * _DLANES, _DLANES)])


_BN = 1000


def _dinv_block(deg_ref):
    return lax.rsqrt(deg_ref[:, 0:1] + deg_ref[:, _DLANES:_DLANES + 1] + 1.0)


_DEG_SPEC = pl.BlockSpec((_BN, 2 * _DLANES), lambda i: (i, 0))


def _tc_matmul(x, W1):
    def body(x_ref, w_ref, h_ref):
        h_ref[...] = jnp.dot(x_ref[...], w_ref[...],
                             preferred_element_type=jnp.float32)

    F = W1.shape[1]
    return pl.pallas_call(
        body,
        grid=(N_NODES // _BN,),
        in_specs=[
            pl.BlockSpec((_BN, 128), lambda i: (i, 0)),
            pl.BlockSpec((128, F), lambda i: (0, 0)),
        ],
        out_specs=pl.BlockSpec((_BN, F), lambda i: (i, 0)),
        out_shape=jax.ShapeDtypeStruct((N_NODES, F), jnp.float32),
    )(x, W1)


def _tc_scale(h, degs):
    def body(h_ref, deg_ref, g_ref):
        g_ref[...] = h_ref[...] * _dinv_block(deg_ref)

    F = h.shape[1]
    return pl.pallas_call(
        body,
        grid=(N_NODES // _BN,),
        in_specs=[
            pl.BlockSpec((_BN, F), lambda i: (i, 0)),
            _DEG_SPEC,
        ],
        out_specs=pl.BlockSpec((_BN, F), lambda i: (i, 0)),
        out_shape=jax.ShapeDtypeStruct((N_NODES, F), jnp.float32),
    )(h, degs)


def _tc_mid(acc, h_prev, degs, b_prev, W):
    Fp, F = W.shape

    def body(a_ref, h_ref, deg_ref, b_ref, w_ref, ho_ref, go_ref):
        d = _dinv_block(deg_ref)
        z = d * a_ref[...] + (d * d) * h_ref[...] + b_ref[...]
        z = jnp.maximum(z, 0.0)
        h = jnp.dot(z, w_ref[...], preferred_element_type=jnp.float32)
        ho_ref[...] = h
        go_ref[...] = h * d

    return pl.pallas_call(
        body,
        grid=(N_NODES // _BN,),
        in_specs=[
            pl.BlockSpec((_BN, Fp), lambda i: (i, 0)),
            pl.BlockSpec((_BN, Fp), lambda i: (i, 0)),
            _DEG_SPEC,
            pl.BlockSpec((1, Fp), lambda i: (0, 0)),
            pl.BlockSpec((Fp, F), lambda i: (0, 0)),
        ],
        out_specs=[
            pl.BlockSpec((_BN, F), lambda i: (i, 0)),
            pl.BlockSpec((_BN, F), lambda i: (i, 0)),
        ],
        out_shape=[jax.ShapeDtypeStruct((N_NODES, F), jnp.float32)] * 2,
    )(acc, h_prev, degs, b_prev, W)


def _tc_last(acc, h_prev, degs, b):
    F = h_prev.shape[1]

    def body(a_ref, h_ref, deg_ref, b_ref, o_ref):
        d = _dinv_block(deg_ref)
        z = d * a_ref[...] + (d * d) * h_ref[...] + b_ref[...]
        m = jnp.max(z, axis=1, keepdims=True)
        e = jnp.exp(z - m)
        lse = jnp.log(jnp.sum(e, axis=1, keepdims=True))
        o_ref[...] = (z - m) - lse

    return pl.pallas_call(
        body,
        grid=(N_NODES // _BN,),
        in_specs=[
            pl.BlockSpec((_BN, F), lambda i: (i, 0)),
            pl.BlockSpec((_BN, F), lambda i: (i, 0)),
            _DEG_SPEC,
            pl.BlockSpec((1, F), lambda i: (0, 0)),
        ],
        out_specs=pl.BlockSpec((_BN, F), lambda i: (i, 0)),
        out_shape=jax.ShapeDtypeStruct((N_NODES, F), jnp.float32),
    )(acc, h_prev, degs, b)


def kernel(x, edge_index, W1, b1, W2, b2, W3, b3, W4, b4):
    ei = edge_index.astype(jnp.int32)
    npad_d = EPAD_D - N_EDGES
    dstp_d = jnp.concatenate(
        [ei[1], jnp.full((npad_d,), PAD_DST, jnp.int32)]).reshape(
        NT, NCHUNK_D, CHUNK)
    npad_a = EPAD_A - N_EDGES
    srcp_a = jnp.concatenate(
        [ei[0], jnp.zeros((npad_a,), jnp.int32)]).reshape(
        NS, NCHUNK_A, CHUNK)
    dstp_a = jnp.concatenate(
        [ei[1], jnp.full((npad_a,), PAD_DST, jnp.int32)]).reshape(
        NS, NCHUNK_A, CHUNK)

    z8 = jnp.zeros((ROWS, 8), jnp.float32)
    z16 = jnp.zeros((ROWS, 16), jnp.float32)
    z32 = jnp.zeros((ROWS, 32), jnp.float32)
    ones = jnp.ones((CHUNK, _DLANES), jnp.float32)

    degs = _sc_deg(dstp_d, ones, z8)
    h1 = _tc_matmul(x, W1)
    g1 = _tc_scale(h1, degs)

    acc1 = _AGG[64](g1, srcp_a, dstp_a, z32)
    h2, g2 = _tc_mid(acc1, h1, degs, b1.reshape(1, -1), W2)
    acc2 = _AGG[32](g2, srcp_a, dstp_a, z16)
    h3, g3 = _tc_mid(acc2, h2, degs, b2.reshape(1, -1), W3)
    acc3 = _AGG[16](g3, srcp_a, dstp_a, z8)
    h4, g4 = _tc_mid(acc3, h3, degs, b3.reshape(1, -1), W4)
    acc4 = _AGG[16](g4, srcp_a, dstp_a, z8)
    return _tc_last(acc4, h4, degs, b4.reshape(1, -1))


# speedup vs baseline: 1.0722x; 1.0722x over previous
import functools

import jax
import jax.numpy as jnp
from jax import lax
from jax.experimental import pallas as pl
from jax.experimental.pallas import tpu as pltpu
from jax.experimental.pallas import tpu_sc as plsc

N_NODES = 10000
N_EDGES = 320000
NC, NS = 2, 16
NT = NC * NS
CHUNK = 128
NCHUNK = 80
EPAD = NT * NCHUNK * CHUNK
ROWS = 10112
RPT = ROWS // NS
NPT = N_NODES // NS
PAD_DST = N_NODES + 8
_DLANES = 8
_MESH = plsc.VectorSubcoreMesh(core_axis_name="c", subcore_axis_name="s")


def _make_sc_agg(F):
    @functools.partial(
        pl.kernel,
        out_type=jax.ShapeDtypeStruct((ROWS, 2 * F), jnp.float32),
        mesh=_MESH,
        compiler_params=pltpu.CompilerParams(use_tc_tiling_on_sc=False),
        scratch_types=[
            pltpu.VMEM((NCHUNK, CHUNK), jnp.int32),
            pltpu.VMEM((NCHUNK, CHUNK), jnp.int32),
            pltpu.VMEM((CHUNK, F), jnp.float32),
            pltpu.VMEM((CHUNK, F), jnp.float32),
            pltpu.VMEM_SHARED((N_NODES, F), jnp.float32),
            pltpu.VMEM_SHARED((ROWS, F), jnp.float32),
            pltpu.SemaphoreType.DMA,
            pltpu.SemaphoreType.DMA,
        ],
    )
    def agg(g_hbm, src_hbm, dst_hbm, zero_hbm, out_hbm,
            src_v, dst_v, gb0, gb1, gsh, acc, sem0, sem1):
        c = lax.axis_index("c")
        s = lax.axis_index("s")
        wid = c * NS + s
        pltpu.sync_copy(src_hbm.at[wid], src_v)
        pltpu.sync_copy(dst_hbm.at[wid], dst_v)
        g0 = s * NPT
        pltpu.sync_copy(g_hbm.at[pl.ds(g0, NPT)], gsh.at[pl.ds(g0, NPT)])
        r0 = s * RPT
        pltpu.sync_copy(zero_hbm.at[pl.ds(r0, RPT)], acc.at[pl.ds(r0, RPT)])
        plsc.subcore_barrier()

        gbufs = (gb0, gb1)
        sems = (sem0, sem1)
        for b in range(2):
            pltpu.async_copy(gsh.at[src_v.at[b]], gbufs[b], sems[b])

        @pl.loop(0, NCHUNK - 2, step=2)
        def _(j0):
            for b in range(2):
                j = j0 + b
                pltpu.make_async_copy(gsh.at[src_v.at[j]], gbufs[b], sems[b]).wait()
                pltpu.sync_copy(gbufs[b], acc.at[dst_v.at[j]], add=True)
                pltpu.async_copy(gsh.at[src_v.at[j + 2]], gbufs[b], sems[b])

        for b in range(2):
            j = NCHUNK - 2 + b
            pltpu.make_async_copy(gsh.at[src_v.at[j]], gbufs[b], sems[b]).wait()
            pltpu.sync_copy(gbufs[b], acc.at[dst_v.at[j]], add=True)

        plsc.subcore_barrier()
        pltpu.sync_copy(acc.at[pl.ds(r0, RPT)],
                        out_hbm.at[pl.ds(r0, RPT), pl.ds(c * F, F)])

    return agg


_AGG = {f: _make_sc_agg(f) for f in (64, 32, 16)}


@functools.partial(
    pl.kernel,
    out_type=jax.ShapeDtypeStruct((ROWS, 2 * _DLANES), jnp.float32),
    mesh=_MESH,
    compiler_params=pltpu.CompilerParams(use_tc_tiling_on_sc=False),
    scratch_types=[
        pltpu.VMEM((NCHUNK, CHUNK), jnp.int32),
        pltpu.VMEM((CHUNK, _DLANES), jnp.float32),
        pltpu.VMEM_SHARED((ROWS, _DLANES), jnp.float32),
        pltpu.SemaphoreType.DMA,
    ],
)
def _sc_deg(dst_hbm, ones_hbm, zero_hbm, out_hbm, dst_v, ones_v, acc, sem):
    c = lax.axis_index("c")
    s = lax.axis_index("s")
    wid = c * NS + s
    pltpu.sync_copy(dst_hbm.at[wid], dst_v)
    pltpu.sync_copy(ones_hbm, ones_v)
    r0 = s * RPT
    pltpu.sync_copy(zero_hbm.at[pl.ds(r0, RPT)], acc.at[pl.ds(r0, RPT)])
    plsc.subcore_barrier()

    @pl.loop(0, NCHUNK)
    def _(j):
        pltpu.async_copy(ones_v, acc.at[dst_v.at[j]], sem, add=True)

    @pl.loop(0, NCHUNK)
    def _(j):
        pltpu.make_async_copy(ones_v, acc.at[dst_v.at[j]], sem).wait()

    plsc.subcore_barrier()
    pltpu.sync_copy(acc.at[pl.ds(r0, RPT)],
                    out_hbm.at[pl.ds(r0, RPT), pl.ds(c * _DLANES, _DLANES)])


_BN = 1000


def _dinv_block(deg_ref):
    return lax.rsqrt(deg_ref[:, 0:1] + deg_ref[:, _DLANES:_DLANES + 1] + 1.0)


_DEG_SPEC = pl.BlockSpec((_BN, 2 * _DLANES), lambda i: (i, 0))


def _tc_matmul(x, W1):
    def body(x_ref, w_ref, h_ref):
        h_ref[...] = jnp.dot(x_ref[...], w_ref[...],
                             preferred_element_type=jnp.float32)

    F = W1.shape[1]
    return pl.pallas_call(
        body,
        grid=(N_NODES // _BN,),
        in_specs=[
            pl.BlockSpec((_BN, 128), lambda i: (i, 0)),
            pl.BlockSpec((128, F), lambda i: (0, 0)),
        ],
        out_specs=pl.BlockSpec((_BN, F), lambda i: (i, 0)),
        out_shape=jax.ShapeDtypeStruct((N_NODES, F), jnp.float32),
    )(x, W1)


def _tc_scale(h, degs):
    def body(h_ref, deg_ref, g_ref):
        g_ref[...] = h_ref[...] * _dinv_block(deg_ref)

    F = h.shape[1]
    return pl.pallas_call(
        body,
        grid=(N_NODES // _BN,),
        in_specs=[
            pl.BlockSpec((_BN, F), lambda i: (i, 0)),
            _DEG_SPEC,
        ],
        out_specs=pl.BlockSpec((_BN, F), lambda i: (i, 0)),
        out_shape=jax.ShapeDtypeStruct((N_NODES, F), jnp.float32),
    )(h, degs)


def _tc_mid(acc, h_prev, degs, b_prev, W):
    Fp, F = W.shape

    def body(a_ref, h_ref, deg_ref, b_ref, w_ref, ho_ref, go_ref):
        d = _dinv_block(deg_ref)
        a = a_ref[:, :Fp] + a_ref[:, Fp:]
        z = d * a + (d * d) * h_ref[...] + b_ref[...]
        z = jnp.maximum(z, 0.0)
        h = jnp.dot(z, w_ref[...], preferred_element_type=jnp.float32)
        ho_ref[...] = h
        go_ref[...] = h * d

    return pl.pallas_call(
        body,
        grid=(N_NODES // _BN,),
        in_specs=[
            pl.BlockSpec((_BN, 2 * Fp), lambda i: (i, 0)),
            pl.BlockSpec((_BN, Fp), lambda i: (i, 0)),
            _DEG_SPEC,
            pl.BlockSpec((1, Fp), lambda i: (0, 0)),
            pl.BlockSpec((Fp, F), lambda i: (0, 0)),
        ],
        out_specs=[
            pl.BlockSpec((_BN, F), lambda i: (i, 0)),
            pl.BlockSpec((_BN, F), lambda i: (i, 0)),
        ],
        out_shape=[jax.ShapeDtypeStruct((N_NODES, F), jnp.float32)] * 2,
    )(acc, h_prev, degs, b_prev, W)


def _tc_last(acc, h_prev, degs, b):
    F = h_prev.shape[1]

    def body(a_ref, h_ref, deg_ref, b_ref, o_ref):
        d = _dinv_block(deg_ref)
        a = a_ref[:, :F] + a_ref[:, F:]
        z = d * a + (d * d) * h_ref[...] + b_ref[...]
        m = jnp.max(z, axis=1, keepdims=True)
        e = jnp.exp(z - m)
        lse = jnp.log(jnp.sum(e, axis=1, keepdims=True))
        o_ref[...] = (z - m) - lse

    return pl.pallas_call(
        body,
        grid=(N_NODES // _BN,),
        in_specs=[
            pl.BlockSpec((_BN, 2 * F), lambda i: (i, 0)),
            pl.BlockSpec((_BN, F), lambda i: (i, 0)),
            _DEG_SPEC,
            pl.BlockSpec((1, F), lambda i: (0, 0)),
        ],
        out_specs=pl.BlockSpec((_BN, F), lambda i: (i, 0)),
        out_shape=jax.ShapeDtypeStruct((N_NODES, F), jnp.float32),
    )(acc, h_prev, degs, b)


def kernel(x, edge_index, W1, b1, W2, b2, W3, b3, W4, b4):
    ei = edge_index.astype(jnp.int32)
    npad = EPAD - N_EDGES
    srcp = jnp.concatenate([ei[0], jnp.zeros((npad,), jnp.int32)]).reshape(
        NT, NCHUNK, CHUNK)
    dstp = jnp.concatenate([ei[1], jnp.full((npad,), PAD_DST, jnp.int32)]).reshape(
        NT, NCHUNK, CHUNK)

    z8 = jnp.zeros((ROWS, 8), jnp.float32)
    z16 = jnp.zeros((ROWS, 16), jnp.float32)
    z32 = jnp.zeros((ROWS, 32), jnp.float32)
    z64 = jnp.zeros((ROWS, 64), jnp.float32)
    ones = jnp.ones((CHUNK, _DLANES), jnp.float32)

    degs = _sc_deg(dstp, ones, z8)
    h1 = _tc_matmul(x, W1)
    g1 = _tc_scale(h1, degs)

    acc1 = _AGG[64](g1, srcp, dstp, z64)
    h2, g2 = _tc_mid(acc1, h1, degs, b1.reshape(1, -1), W2)
    acc2 = _AGG[32](g2, srcp, dstp, z32)
    h3, g3 = _tc_mid(acc2, h2, degs, b2.reshape(1, -1), W3)
    acc3 = _AGG[16](g3, srcp, dstp, z16)
    h4, g4 = _tc_mid(acc3, h3, degs, b3.reshape(1, -1), W4)
    acc4 = _AGG[16](g4, srcp, dstp, z16)
    return _tc_last(acc4, h4, degs, b4.reshape(1, -1))


# speedup vs baseline: 1.1034x; 1.0291x over previous
import functools

import jax
import jax.numpy as jnp
from jax import lax
from jax.experimental import pallas as pl
from jax.experimental.pallas import tpu as pltpu
from jax.experimental.pallas import tpu_sc as plsc

N_NODES = 10000
N_EDGES = 320000
NC, NS = 2, 16
NT = NC * NS
CHUNK = 128
NCHUNK = 80
EPAD = NT * NCHUNK * CHUNK
ROWS = 10112
RPT = ROWS // NS
NPT = N_NODES // NS
PAD_DST = N_NODES + 8
_DLANES = 8
_MESH = plsc.VectorSubcoreMesh(core_axis_name="c", subcore_axis_name="s")


def _make_sc_agg(F):
    @functools.partial(
        pl.kernel,
        out_type=jax.ShapeDtypeStruct((ROWS, 2 * F), jnp.float32),
        mesh=_MESH,
        compiler_params=pltpu.CompilerParams(use_tc_tiling_on_sc=False),
        scratch_types=[
            pltpu.VMEM((NCHUNK, CHUNK), jnp.int32),
            pltpu.VMEM((NCHUNK, CHUNK), jnp.int32),
            pltpu.VMEM((CHUNK, F), jnp.float32),
            pltpu.VMEM((CHUNK, F), jnp.float32),
            pltpu.VMEM_SHARED((N_NODES, F), jnp.float32),
            pltpu.VMEM_SHARED((ROWS, F), jnp.float32),
            pltpu.SemaphoreType.DMA,
            pltpu.SemaphoreType.DMA,
        ],
    )
    def agg(g_hbm, src_hbm, dst_hbm, zero_hbm, out_hbm,
            src_v, dst_v, gb0, gb1, gsh, acc, sem0, sem1):
        c = lax.axis_index("c")
        s = lax.axis_index("s")
        wid = c * NS + s
        pltpu.sync_copy(src_hbm.at[wid], src_v)
        pltpu.sync_copy(dst_hbm.at[wid], dst_v)
        g0 = s * NPT
        pltpu.sync_copy(g_hbm.at[pl.ds(g0, NPT)], gsh.at[pl.ds(g0, NPT)])
        r0 = s * RPT
        pltpu.sync_copy(zero_hbm.at[pl.ds(r0, RPT)], acc.at[pl.ds(r0, RPT)])
        plsc.subcore_barrier()

        gbufs = (gb0, gb1)
        sems = (sem0, sem1)
        for b in range(2):
            pltpu.async_copy(gsh.at[src_v.at[b]], gbufs[b], sems[b])

        @pl.loop(0, NCHUNK - 2, step=2)
        def _(j0):
            for b in range(2):
                j = j0 + b
                pltpu.make_async_copy(gsh.at[src_v.at[j]], gbufs[b], sems[b]).wait()
                pltpu.sync_copy(gbufs[b], acc.at[dst_v.at[j]], add=True)
                pltpu.async_copy(gsh.at[src_v.at[j + 2]], gbufs[b], sems[b])

        for b in range(2):
            j = NCHUNK - 2 + b
            pltpu.make_async_copy(gsh.at[src_v.at[j]], gbufs[b], sems[b]).wait()
            pltpu.sync_copy(gbufs[b], acc.at[dst_v.at[j]], add=True)

        plsc.subcore_barrier()
        pltpu.sync_copy(acc.at[pl.ds(r0, RPT)],
                        out_hbm.at[pl.ds(r0, RPT), pl.ds(c * F, F)])

    return agg


_AGG = {f: _make_sc_agg(f) for f in (64, 32, 16)}


@functools.partial(
    pl.kernel,
    out_type=jax.ShapeDtypeStruct((ROWS, 2 * _DLANES), jnp.float32),
    mesh=_MESH,
    compiler_params=pltpu.CompilerParams(use_tc_tiling_on_sc=False),
    scratch_types=[
        pltpu.VMEM((NCHUNK, CHUNK), jnp.int32),
        pltpu.VMEM((CHUNK, _DLANES), jnp.float32),
        pltpu.VMEM_SHARED((ROWS, _DLANES), jnp.float32),
        pltpu.SemaphoreType.DMA,
    ],
)
def _sc_deg(dst_hbm, ones_hbm, zero_hbm, out_hbm, dst_v, ones_v, acc, sem):
    c = lax.axis_index("c")
    s = lax.axis_index("s")
    wid = c * NS + s
    pltpu.sync_copy(dst_hbm.at[wid], dst_v)
    pltpu.sync_copy(ones_hbm, ones_v)
    r0 = s * RPT
    pltpu.sync_copy(zero_hbm.at[pl.ds(r0, RPT)], acc.at[pl.ds(r0, RPT)])
    plsc.subcore_barrier()

    @pl.loop(0, NCHUNK)
    def _(j):
        pltpu.async_copy(ones_v, acc.at[dst_v.at[j]], sem, add=True)

    @pl.loop(0, NCHUNK)
    def _(j):
        pltpu.make_async_copy(ones_v, acc.at[dst_v.at[j]], sem).wait()

    plsc.subcore_barrier()
    pltpu.sync_copy(acc.at[pl.ds(r0, RPT)],
                    out_hbm.at[pl.ds(r0, RPT), pl.ds(c * _DLANES, _DLANES)])


_BN = 2000


def _dinv_block(deg_ref):
    return lax.rsqrt(deg_ref[:, 0:1] + deg_ref[:, _DLANES:_DLANES + 1] + 1.0)


_DEG_SPEC = pl.BlockSpec((_BN, 2 * _DLANES), lambda i: (i, 0))


def _tc_matmul(x, W1):
    def body(x_ref, w_ref, h_ref):
        h_ref[...] = jnp.dot(x_ref[...], w_ref[...],
                             preferred_element_type=jnp.float32)

    F = W1.shape[1]
    return pl.pallas_call(
        body,
        grid=(N_NODES // _BN,),
        in_specs=[
            pl.BlockSpec((_BN, 128), lambda i: (i, 0)),
            pl.BlockSpec((128, F), lambda i: (0, 0)),
        ],
        out_specs=pl.BlockSpec((_BN, F), lambda i: (i, 0)),
        out_shape=jax.ShapeDtypeStruct((N_NODES, F), jnp.float32),
    )(x, W1)


def _tc_scale(h, degs):
    def body(h_ref, deg_ref, g_ref):
        g_ref[...] = h_ref[...] * _dinv_block(deg_ref)

    F = h.shape[1]
    return pl.pallas_call(
        body,
        grid=(N_NODES // _BN,),
        in_specs=[
            pl.BlockSpec((_BN, F), lambda i: (i, 0)),
            _DEG_SPEC,
        ],
        out_specs=pl.BlockSpec((_BN, F), lambda i: (i, 0)),
        out_shape=jax.ShapeDtypeStruct((N_NODES, F), jnp.float32),
    )(h, degs)


def _tc_mid(acc, h_prev, degs, b_prev, W):
    Fp, F = W.shape

    def body(a_ref, h_ref, deg_ref, b_ref, w_ref, ho_ref, go_ref):
        d = _dinv_block(deg_ref)
        a = a_ref[:, :Fp] + a_ref[:, Fp:]
        z = d * a + (d * d) * h_ref[...] + b_ref[...]
        z = jnp.maximum(z, 0.0)
        h = jnp.dot(z, w_ref[...], preferred_element_type=jnp.float32)
        ho_ref[...] = h
        go_ref[...] = h * d

    return pl.pallas_call(
        body,
        grid=(N_NODES // _BN,),
        in_specs=[
            pl.BlockSpec((_BN, 2 * Fp), lambda i: (i, 0)),
            pl.BlockSpec((_BN, Fp), lambda i: (i, 0)),
            _DEG_SPEC,
            pl.BlockSpec((1, Fp), lambda i: (0, 0)),
            pl.BlockSpec((Fp, F), lambda i: (0, 0)),
        ],
        out_specs=[
            pl.BlockSpec((_BN, F), lambda i: (i, 0)),
            pl.BlockSpec((_BN, F), lambda i: (i, 0)),
        ],
        out_shape=[jax.ShapeDtypeStruct((N_NODES, F), jnp.float32)] * 2,
    )(acc, h_prev, degs, b_prev, W)


def _tc_last(acc, h_prev, degs, b):
    F = h_prev.shape[1]

    def body(a_ref, h_ref, deg_ref, b_ref, o_ref):
        d = _dinv_block(deg_ref)
        a = a_ref[:, :F] + a_ref[:, F:]
        z = d * a + (d * d) * h_ref[...] + b_ref[...]
        m = jnp.max(z, axis=1, keepdims=True)
        e = jnp.exp(z - m)
        lse = jnp.log(jnp.sum(e, axis=1, keepdims=True))
        o_ref[...] = (z - m) - lse

    return pl.pallas_call(
        body,
        grid=(N_NODES // _BN,),
        in_specs=[
            pl.BlockSpec((_BN, 2 * F), lambda i: (i, 0)),
            pl.BlockSpec((_BN, F), lambda i: (i, 0)),
            _DEG_SPEC,
            pl.BlockSpec((1, F), lambda i: (0, 0)),
        ],
        out_specs=pl.BlockSpec((_BN, F), lambda i: (i, 0)),
        out_shape=jax.ShapeDtypeStruct((N_NODES, F), jnp.float32),
    )(acc, h_prev, degs, b)


def kernel(x, edge_index, W1, b1, W2, b2, W3, b3, W4, b4):
    ei = edge_index.astype(jnp.int32)
    npad = EPAD - N_EDGES
    srcp = jnp.concatenate([ei[0], jnp.zeros((npad,), jnp.int32)]).reshape(
        NT, NCHUNK, CHUNK)
    dstp = jnp.concatenate([ei[1], jnp.full((npad,), PAD_DST, jnp.int32)]).reshape(
        NT, NCHUNK, CHUNK)

    z8 = jnp.zeros((ROWS, 8), jnp.float32)
    z16 = jnp.zeros((ROWS, 16), jnp.float32)
    z32 = jnp.zeros((ROWS, 32), jnp.float32)
    z64 = jnp.zeros((ROWS, 64), jnp.float32)
    ones = jnp.ones((CHUNK, _DLANES), jnp.float32)

    degs = _sc_deg(dstp, ones, z8)
    h1 = _tc_matmul(x, W1)
    g1 = _tc_scale(h1, degs)

    acc1 = _AGG[64](g1, srcp, dstp, z64)
    h2, g2 = _tc_mid(acc1, h1, degs, b1.reshape(1, -1), W2)
    acc2 = _AGG[32](g2, srcp, dstp, z32)
    h3, g3 = _tc_mid(acc2, h2, degs, b2.reshape(1, -1), W3)
    acc3 = _AGG[16](g3, srcp, dstp, z16)
    h4, g4 = _tc_mid(acc3, h3, degs, b3.reshape(1, -1), W4)
    acc4 = _AGG[16](g4, srcp, dstp, z16)
    return _tc_last(acc4, h4, degs, b4.reshape(1, -1))


# speedup vs baseline: 1.1104x; 1.0063x over previous
import functools

import jax
import jax.numpy as jnp
from jax import lax
from jax.experimental import pallas as pl
from jax.experimental.pallas import tpu as pltpu
from jax.experimental.pallas import tpu_sc as plsc

N_NODES = 10000
N_EDGES = 320000
NC, NS = 2, 16
NT = NC * NS
CHUNK = 128
NCHUNK = 80
EPAD = NT * NCHUNK * CHUNK
ROWS = 10112
RPT = ROWS // NS
NPT = N_NODES // NS
PAD_DST = N_NODES + 8
_DLANES = 8
_MESH = plsc.VectorSubcoreMesh(core_axis_name="c", subcore_axis_name="s")


def _make_sc_agg(F):
    @functools.partial(
        pl.kernel,
        out_type=jax.ShapeDtypeStruct((ROWS, 2 * F), jnp.float32),
        mesh=_MESH,
        compiler_params=pltpu.CompilerParams(use_tc_tiling_on_sc=False),
        scratch_types=[
            pltpu.VMEM((NCHUNK, CHUNK), jnp.int32),
            pltpu.VMEM((NCHUNK, CHUNK), jnp.int32),
            pltpu.VMEM((CHUNK, F), jnp.float32),
            pltpu.VMEM((CHUNK, F), jnp.float32),
            pltpu.VMEM_SHARED((N_NODES, F), jnp.float32),
            pltpu.VMEM_SHARED((ROWS, F), jnp.float32),
            pltpu.SemaphoreType.DMA,
            pltpu.SemaphoreType.DMA,
        ],
    )
    def agg(g_hbm, src_hbm, dst_hbm, zero_hbm, out_hbm,
            src_v, dst_v, gb0, gb1, gsh, acc, sem0, sem1):
        c = lax.axis_index("c")
        s = lax.axis_index("s")
        wid = c * NS + s
        pltpu.sync_copy(src_hbm.at[wid], src_v)
        pltpu.sync_copy(dst_hbm.at[wid], dst_v)
        g0 = s * NPT
        pltpu.sync_copy(g_hbm.at[pl.ds(g0, NPT)], gsh.at[pl.ds(g0, NPT)])
        r0 = s * RPT
        pltpu.sync_copy(zero_hbm.at[pl.ds(r0, RPT)], acc.at[pl.ds(r0, RPT)])
        plsc.subcore_barrier()

        gbufs = (gb0, gb1)
        sems = (sem0, sem1)
        for b in range(2):
            pltpu.async_copy(gsh.at[src_v.at[b]], gbufs[b], sems[b])

        @pl.loop(0, NCHUNK - 2, step=2)
        def _(j0):
            for b in range(2):
                j = j0 + b
                pltpu.make_async_copy(gsh.at[src_v.at[j]], gbufs[b], sems[b]).wait()
                pltpu.sync_copy(gbufs[b], acc.at[dst_v.at[j]], add=True)
                pltpu.async_copy(gsh.at[src_v.at[j + 2]], gbufs[b], sems[b])

        for b in range(2):
            j = NCHUNK - 2 + b
            pltpu.make_async_copy(gsh.at[src_v.at[j]], gbufs[b], sems[b]).wait()
            pltpu.sync_copy(gbufs[b], acc.at[dst_v.at[j]], add=True)

        plsc.subcore_barrier()
        pltpu.sync_copy(acc.at[pl.ds(r0, RPT)],
                        out_hbm.at[pl.ds(r0, RPT), pl.ds(c * F, F)])

    return agg


_AGG = {f: _make_sc_agg(f) for f in (64, 32, 16)}


@functools.partial(
    pl.kernel,
    out_type=jax.ShapeDtypeStruct((ROWS, 2 * _DLANES), jnp.float32),
    mesh=_MESH,
    compiler_params=pltpu.CompilerParams(use_tc_tiling_on_sc=False),
    scratch_types=[
        pltpu.VMEM((NCHUNK, CHUNK), jnp.int32),
        pltpu.VMEM((CHUNK, _DLANES), jnp.float32),
        pltpu.VMEM_SHARED((ROWS, _DLANES), jnp.float32),
        pltpu.SemaphoreType.DMA,
    ],
)
def _sc_deg(dst_hbm, ones_hbm, zero_hbm, out_hbm, dst_v, ones_v, acc, sem):
    c = lax.axis_index("c")
    s = lax.axis_index("s")
    wid = c * NS + s
    pltpu.sync_copy(dst_hbm.at[wid], dst_v)
    pltpu.sync_copy(ones_hbm, ones_v)
    r0 = s * RPT
    pltpu.sync_copy(zero_hbm.at[pl.ds(r0, RPT)], acc.at[pl.ds(r0, RPT)])
    plsc.subcore_barrier()

    @pl.loop(0, NCHUNK)
    def _(j):
        pltpu.async_copy(ones_v, acc.at[dst_v.at[j]], sem, add=True)

    @pl.loop(0, NCHUNK)
    def _(j):
        pltpu.make_async_copy(ones_v, acc.at[dst_v.at[j]], sem).wait()

    plsc.subcore_barrier()
    pltpu.sync_copy(acc.at[pl.ds(r0, RPT)],
                    out_hbm.at[pl.ds(r0, RPT), pl.ds(c * _DLANES, _DLANES)])


_BN = 5000


def _dinv_block(deg_ref):
    return lax.rsqrt(deg_ref[:, 0:1] + deg_ref[:, _DLANES:_DLANES + 1] + 1.0)


_DEG_SPEC = pl.BlockSpec((_BN, 2 * _DLANES), lambda i: (i, 0))


def _tc_matmul(x, W1):
    def body(x_ref, w_ref, h_ref):
        h_ref[...] = jnp.dot(x_ref[...], w_ref[...],
                             preferred_element_type=jnp.float32)

    F = W1.shape[1]
    return pl.pallas_call(
        body,
        grid=(N_NODES // _BN,),
        in_specs=[
            pl.BlockSpec((_BN, 128), lambda i: (i, 0)),
            pl.BlockSpec((128, F), lambda i: (0, 0)),
        ],
        out_specs=pl.BlockSpec((_BN, F), lambda i: (i, 0)),
        out_shape=jax.ShapeDtypeStruct((N_NODES, F), jnp.float32),
    )(x, W1)


def _tc_scale(h, degs):
    def body(h_ref, deg_ref, g_ref):
        g_ref[...] = h_ref[...] * _dinv_block(deg_ref)

    F = h.shape[1]
    return pl.pallas_call(
        body,
        grid=(N_NODES // _BN,),
        in_specs=[
            pl.BlockSpec((_BN, F), lambda i: (i, 0)),
            _DEG_SPEC,
        ],
        out_specs=pl.BlockSpec((_BN, F), lambda i: (i, 0)),
        out_shape=jax.ShapeDtypeStruct((N_NODES, F), jnp.float32),
    )(h, degs)


def _tc_mid(acc, h_prev, degs, b_prev, W):
    Fp, F = W.shape

    def body(a_ref, h_ref, deg_ref, b_ref, w_ref, ho_ref, go_ref):
        d = _dinv_block(deg_ref)
        a = a_ref[:, :Fp] + a_ref[:, Fp:]
        z = d * a + (d * d) * h_ref[...] + b_ref[...]
        z = jnp.maximum(z, 0.0)
        h = jnp.dot(z, w_ref[...], preferred_element_type=jnp.float32)
        ho_ref[...] = h
        go_ref[...] = h * d

    return pl.pallas_call(
        body,
        grid=(N_NODES // _BN,),
        in_specs=[
            pl.BlockSpec((_BN, 2 * Fp), lambda i: (i, 0)),
            pl.BlockSpec((_BN, Fp), lambda i: (i, 0)),
            _DEG_SPEC,
            pl.BlockSpec((1, Fp), lambda i: (0, 0)),
            pl.BlockSpec((Fp, F), lambda i: (0, 0)),
        ],
        out_specs=[
            pl.BlockSpec((_BN, F), lambda i: (i, 0)),
            pl.BlockSpec((_BN, F), lambda i: (i, 0)),
        ],
        out_shape=[jax.ShapeDtypeStruct((N_NODES, F), jnp.float32)] * 2,
    )(acc, h_prev, degs, b_prev, W)


def _tc_last(acc, h_prev, degs, b):
    F = h_prev.shape[1]

    def body(a_ref, h_ref, deg_ref, b_ref, o_ref):
        d = _dinv_block(deg_ref)
        a = a_ref[:, :F] + a_ref[:, F:]
        z = d * a + (d * d) * h_ref[...] + b_ref[...]
        m = jnp.max(z, axis=1, keepdims=True)
        e = jnp.exp(z - m)
        lse = jnp.log(jnp.sum(e, axis=1, keepdims=True))
        o_ref[...] = (z - m) - lse

    return pl.pallas_call(
        body,
        grid=(N_NODES // _BN,),
        in_specs=[
            pl.BlockSpec((_BN, 2 * F), lambda i: (i, 0)),
            pl.BlockSpec((_BN, F), lambda i: (i, 0)),
            _DEG_SPEC,
            pl.BlockSpec((1, F), lambda i: (0, 0)),
        ],
        out_specs=pl.BlockSpec((_BN, F), lambda i: (i, 0)),
        out_shape=jax.ShapeDtypeStruct((N_NODES, F), jnp.float32),
    )(acc, h_prev, degs, b)


def kernel(x, edge_index, W1, b1, W2, b2, W3, b3, W4, b4):
    ei = edge_index.astype(jnp.int32)
    npad = EPAD - N_EDGES
    srcp = jnp.concatenate([ei[0], jnp.zeros((npad,), jnp.int32)]).reshape(
        NT, NCHUNK, CHUNK)
    dstp = jnp.concatenate([ei[1], jnp.full((npad,), PAD_DST, jnp.int32)]).reshape(
        NT, NCHUNK, CHUNK)

    z8 = jnp.zeros((ROWS, 8), jnp.float32)
    z16 = jnp.zeros((ROWS, 16), jnp.float32)
    z32 = jnp.zeros((ROWS, 32), jnp.float32)
    z64 = jnp.zeros((ROWS, 64), jnp.float32)
    ones = jnp.ones((CHUNK, _DLANES), jnp.float32)

    degs = _sc_deg(dstp, ones, z8)
    h1 = _tc_matmul(x, W1)
    g1 = _tc_scale(h1, degs)

    acc1 = _AGG[64](g1, srcp, dstp, z64)
    h2, g2 = _tc_mid(acc1, h1, degs, b1.reshape(1, -1), W2)
    acc2 = _AGG[32](g2, srcp, dstp, z32)
    h3, g3 = _tc_mid(acc2, h2, degs, b2.reshape(1, -1), W3)
    acc3 = _AGG[16](g3, srcp, dstp, z16)
    h4, g4 = _tc_mid(acc3, h3, degs, b3.reshape(1, -1), W4)
    acc4 = _AGG[16](g4, srcp, dstp, z16)
    return _tc_last(acc4, h4, degs, b4.reshape(1, -1))
